# baseline (speedup 1.0000x reference)
import functools
import math

import jax
import jax.numpy as jnp
from jax import lax
from jax.experimental import pallas as pl
from jax.experimental.pallas import tpu as pltpu
from jax.experimental.pallas import tpu_sc as plsc

V = 32000
SMOOTH_VAL = 0.1 / (V - 2)
CONFIDENCE = 0.9
C1 = SMOOTH_VAL * (V - 2) * math.log(SMOOTH_VAL) + CONFIDENCE * math.log(CONFIDENCE)

BR = 128
BC = 32000

NW = 32
RSC = 2048


def _tc_body(tgt_ref, out_ref, acc_ref):
    i = pl.program_id(0)
    j = pl.program_id(1)

    @pl.when((i == 0) & (j == 0))
    def _init():
        acc_ref[0, 0] = 0.0

    blk = out_ref[...]
    m = (tgt_ref[...] != 0).astype(jnp.float32)
    rs = jnp.sum(blk, axis=1, keepdims=True)
    part = -SMOOTH_VAL * jnp.sum(rs * m)
    extra = jnp.sum(m * (C1 + SMOOTH_VAL * blk[:, 0:1]))
    part = part + jnp.where(j == 0, extra, 0.0)
    acc_ref[0, 0] += part


def _tc_partial(out2d, tgt2d):
    n = out2d.shape[0]
    rb0 = RSC // BR
    return pl.pallas_call(
        _tc_body,
        grid=((n - RSC) // BR, V // BC),
        in_specs=[
            pl.BlockSpec((BR, 1), lambda i, j: (i + rb0, 0)),
            pl.BlockSpec((BR, BC), lambda i, j: (i + rb0, j)),
        ],
        out_specs=pl.BlockSpec(
            (1, 1), lambda i, j: (0, 0), memory_space=pltpu.SMEM),
        out_shape=jax.ShapeDtypeStruct((1, 1), jnp.float32),
    )(tgt2d, out2d)


def _sc_partial(outflat, tgt):
    n = tgt.shape[0]
    gch = n // NW
    rpw = RSC // NW
    mesh = plsc.VectorSubcoreMesh(core_axis_name="c", subcore_axis_name="s")

    @functools.partial(
        pl.kernel,
        mesh=mesh,
        out_type=jax.ShapeDtypeStruct((NW, 16), jnp.float32),
        scratch_types=[
            pltpu.VMEM((gch,), jnp.int32),
            pltpu.VMEM((rpw,), jnp.int32),
            pltpu.VMEM((gch,), jnp.int32),
            pltpu.VMEM((gch,), jnp.float32),
            pltpu.VMEM((V,), jnp.float32),
            pltpu.VMEM((V,), jnp.float32),
            pltpu.VMEM((16,), jnp.float32),
            pltpu.SemaphoreType.DMA,
            pltpu.SemaphoreType.DMA,
            pltpu.SemaphoreType.DMA,
        ],
    )
    def k(outflat_hbm, tgt_hbm, o_hbm,
          gt_v, td_v, idx_v, vals_v, buf0, buf1, res_v, gsem, sem0, sem1):
        wid = lax.axis_index("s") * 2 + lax.axis_index("c")

        gbase = wid * gch
        pltpu.sync_copy(tgt_hbm.at[pl.ds(gbase, gch)], gt_v)
        for q in range(gch // 16):
            t16 = gt_v[pl.ds(q * 16, 16)]
            rows = (gbase + q * 16) + lax.iota(jnp.int32, 16)
            idx_v[pl.ds(q * 16, 16)] = rows * V + t16
        gh = pltpu.async_copy(outflat_hbm.at[idx_v], vals_v, gsem)

        row0 = wid * rpw
        pltpu.sync_copy(tgt_hbm.at[pl.ds(row0, rpw)], td_v)
        bufs = (buf0, buf1)
        sems = (sem0, sem1)
        handles = [None, None]
        handles[0] = pltpu.async_copy(
            outflat_hbm.at[pl.ds(row0 * V, V)], buf0, sem0)

        e0 = jnp.where(lax.iota(jnp.int32, 16) == 0,
                       jnp.float32(1.0), jnp.float32(0.0))
        U = 8
        res_v[...] = jnp.zeros((16,), jnp.float32)
        for r in range(rpw):
            if r + 1 < rpw:
                handles[(r + 1) % 2] = pltpu.async_copy(
                    outflat_hbm.at[pl.ds((row0 + r + 1) * V, V)],
                    bufs[(r + 1) % 2], sems[(r + 1) % 2])
            handles[r % 2].wait()
            b = bufs[r % 2]

            def body(i, accs, b=b):
                base = i * (16 * U)
                return tuple(
                    a + b[pl.ds(base + u * 16, 16)]
                    for u, a in enumerate(accs))
            accs = lax.fori_loop(
                0, V // (16 * U), body,
                tuple(jnp.zeros((16,), jnp.float32) for _ in range(U)))
            svec = accs[0]
            for a in accs[1:]:
                svec = svec + a
            rowvec = (C1 * e0 + SMOOTH_VAL * (b[pl.ds(0, 16)] * e0)
                      - SMOOTH_VAL * svec)
            tval = td_v[pl.ds((r // 16) * 16, 16)][r % 16]

            @pl.when(tval != 0)
            def _acc(rowvec=rowvec):
                res_v[...] = res_v[...] + rowvec

        gh.wait()
        gacc = jnp.zeros((16,), jnp.float32)
        for q in range(gch // 16):
            t16 = gt_v[pl.ds(q * 16, 16)]
            v16 = vals_v[pl.ds(q * 16, 16)]
            gacc = gacc + jnp.where(t16 != 0, v16, 0.0)
        res_v[...] = res_v[...] + (SMOOTH_VAL - CONFIDENCE) * gacc
        pltpu.sync_copy(res_v, o_hbm.at[wid])

    return k(outflat, tgt)


def kernel(output, target, one_hot):
    n = output.shape[0] * output.shape[1]
    out2d = output.reshape(n, V)
    tgt = target.reshape(n).astype(jnp.int32)
    acc = _tc_partial(out2d, tgt.reshape(n, 1))
    g = _sc_partial(output.reshape(-1), tgt)
    return acc[0, 0] + jnp.sum(g)


# speedup vs baseline: 3.4302x; 3.4302x over previous
import functools
import math

import jax
import jax.numpy as jnp
from jax import lax
from jax.experimental import pallas as pl
from jax.experimental.pallas import tpu as pltpu
from jax.experimental.pallas import tpu_sc as plsc

V = 32000
SMOOTH_VAL = 0.1 / (V - 2)
CONFIDENCE = 0.9
C1 = SMOOTH_VAL * (V - 2) * math.log(SMOOTH_VAL) + CONFIDENCE * math.log(CONFIDENCE)

BR = 128
BC = 32000

NW = 32
RSC = 0


def _tc_body(tgt_ref, out_ref, acc_ref):
    i = pl.program_id(0)
    j = pl.program_id(1)

    @pl.when((i == 0) & (j == 0))
    def _init():
        acc_ref[0, 0] = 0.0

    blk = out_ref[...]
    tcol = tgt_ref[...]
    m = (tcol != 0).astype(jnp.float32)
    rs = jnp.sum(blk, axis=1, keepdims=True)
    col = lax.broadcasted_iota(jnp.int32, (BR, BC), 1) + j * BC
    gv = jnp.sum(jnp.where(col == tcol, blk, 0.0), axis=1, keepdims=True)
    part = jnp.sum(m * (-SMOOTH_VAL * rs + (SMOOTH_VAL - CONFIDENCE) * gv))
    extra = jnp.sum(m * (C1 + SMOOTH_VAL * blk[:, 0:1]))
    part = part + jnp.where(j == 0, extra, 0.0)
    acc_ref[0, 0] += part


def _tc_partial(out2d, tgt2d):
    n = out2d.shape[0]
    rb0 = RSC // BR
    return pl.pallas_call(
        _tc_body,
        grid=((n - RSC) // BR, V // BC),
        in_specs=[
            pl.BlockSpec((BR, 1), lambda i, j: (i + rb0, 0)),
            pl.BlockSpec((BR, BC), lambda i, j: (i + rb0, j)),
        ],
        out_specs=pl.BlockSpec(
            (1, 1), lambda i, j: (0, 0), memory_space=pltpu.SMEM),
        out_shape=jax.ShapeDtypeStruct((1, 1), jnp.float32),
    )(tgt2d, out2d)


def kernel(output, target, one_hot):
    n = output.shape[0] * output.shape[1]
    out2d = output.reshape(n, V)
    tgt = target.reshape(n).astype(jnp.int32)
    acc = _tc_partial(out2d, tgt.reshape(n, 1))
    return acc[0, 0]
